# Initial kernel scaffold; baseline (speedup 1.0000x reference)
import jax
import jax.numpy as jnp
from jax.experimental import pallas as pl


def kernel(input_ids, E0, E1, E2, E3, W1, W2, W3):
    raise NotImplementedError("write your pallas kernel here")


# speedup vs baseline: 3.1557x; 3.1557x over previous
import functools

import jax
import jax.numpy as jnp
from jax import lax
from jax.experimental import pallas as pl
from jax.experimental.pallas import tpu as pltpu
from jax.experimental.pallas import tpu_sc as plsc

NC, NS, L = 2, 16, 16
NW = NC * NS
T = 8192
TPW = T // NW
CAP = TPW
XROWS = T + L
CUT = (0, 20000, 40000, 200000, 1000000)
DW = (1024, 512, 256, 128)
GB = (16, 32, 32, 64)
D_OUT = 1024
R = 512
NT = T // R


def _sc_body(ids_hbm, e0, e1, e2, e3, x_out,
             idsv, li0, li1, li2, li3, lp0, lp1, lp2, lp3,
             gb0, gb1, gb2, gb3, sem_g, sem_s):
    lidx = (li0, li1, li2, li3)
    lpos = (lp0, lp1, lp2, lp3)
    tables = (e0, e1, e2, e3)
    bufs = (gb0, gb1, gb2, gb3)
    wid = lax.axis_index("s") * NC + lax.axis_index("c")
    base = wid * TPW
    pltpu.sync_copy(ids_hbm.at[pl.ds(base, TPW)], idsv)

    zeros = jnp.zeros((L,), jnp.int32)
    dummy = jnp.full((L,), T, jnp.int32)
    for j in range(CAP // L):
        for b in range(4):
            lidx[b][pl.ds(j * L, L)] = zeros
            lpos[b][pl.ds(j * L, L)] = dummy

    lane = lax.iota(jnp.int32, L)

    def cbody(j, ps):
        v = idsv[pl.ds(j * L, L)]
        pos = lane + (base + j * L)
        masks = (
            v < CUT[1],
            (v >= CUT[1]) & (v < CUT[2]),
            (v >= CUT[2]) & (v < CUT[3]),
            v >= CUT[3],
        )
        locs = (v, v - CUT[1], v - CUT[2], v - CUT[3])
        out_ps = []
        for b in range(4):
            mi = masks[b].astype(jnp.int32)
            c = plsc.cumsum(mi)
            dst = (c - mi) + ps[b]
            plsc.store_scatter(lidx[b], [dst], locs[b], mask=masks[b])
            plsc.store_scatter(lpos[b], [dst], pos, mask=masks[b])
            out_ps.append(ps[b] + jnp.sum(mi))
        return tuple(out_ps)

    z = jnp.int32(0)
    ps = lax.fori_loop(0, TPW // L, cbody, (z, z, z, z))

    for b in range(4):
        G, d, buf = GB[b], DW[b], bufs[b]
        n = (ps[b] + (G - 1)) // G

        def _slot(i):
            return lax.rem(i, 2) * G

        def _gather_desc(i, b=b, G=G, buf=buf):
            src = tables[b].at[lidx[b].at[pl.ds(i * G, G)]]
            return pltpu.make_async_copy(src, buf.at[pl.ds(_slot(i), G)],
                                         sem_g)

        def _scatter_descs(i, b=b, G=G, d=d, buf=buf):
            descs = []
            for j in range(G // L):
                pos_v = lpos[b][pl.ds(i * G + j * L, L)]
                if d == D_OUT:
                    dst = x_out.at[pos_v]
                else:
                    dst = x_out.at[pos_v, pl.ds(0, d)]
                descs.append(pltpu.make_async_copy(
                    buf.at[pl.ds(_slot(i) + j * L, L)], dst, sem_s))
            return descs

        @pl.when(n > 0)
        def _():
            _gather_desc(0).start()

        def lbody(i, _, n=n, _gather_desc=_gather_desc,
                  _scatter_descs=_scatter_descs):
            _gather_desc(i).wait()

            @pl.when(i >= 1)
            def _():
                for dsc in _scatter_descs(i - 1):
                    dsc.wait()

            for dsc in _scatter_descs(i):
                dsc.start()

            @pl.when(i + 1 < n)
            def _():
                _gather_desc(i + 1).start()

            return 0

        lax.fori_loop(0, n, lbody, 0)

        @pl.when(n > 0)
        def _(n=n, _scatter_descs=_scatter_descs):
            for dsc in _scatter_descs(n - 1):
                dsc.wait()


def _sc_gather(ids, e0, e1, e2, e3):
    mesh = plsc.VectorSubcoreMesh(core_axis_name="c", subcore_axis_name="s",
                                  num_cores=NC, num_subcores=NS)
    f = pl.kernel(
        _sc_body,
        out_type=jax.ShapeDtypeStruct((XROWS, D_OUT), jnp.float32),
        mesh=mesh,
        compiler_params=pltpu.CompilerParams(needs_layout_passes=False),
        scratch_types=[
            pltpu.VMEM((TPW,), jnp.int32),
            pltpu.VMEM((CAP,), jnp.int32),
            pltpu.VMEM((CAP,), jnp.int32),
            pltpu.VMEM((CAP,), jnp.int32),
            pltpu.VMEM((CAP,), jnp.int32),
            pltpu.VMEM((CAP,), jnp.int32),
            pltpu.VMEM((CAP,), jnp.int32),
            pltpu.VMEM((CAP,), jnp.int32),
            pltpu.VMEM((CAP,), jnp.int32),
            pltpu.VMEM((2 * GB[0], DW[0]), jnp.float32),
            pltpu.VMEM((2 * GB[1], DW[1]), jnp.float32),
            pltpu.VMEM((2 * GB[2], DW[2]), jnp.float32),
            pltpu.VMEM((2 * GB[3], DW[3]), jnp.float32),
            pltpu.SemaphoreType.DMA,
            pltpu.SemaphoreType.DMA,
        ],
    )
    return f(ids, e0, e1, e2, e3)


def _tc_body(ids_ref, x_ref, w1_ref, w2_ref, w3_ref, o_ref):
    v = ids_ref[0, 0, :][:, None]
    x = x_ref[...]
    acc = jnp.where(v < CUT[1], x, 0.0)
    m1 = (v >= CUT[1]) & (v < CUT[2])
    acc += jnp.dot(jnp.where(m1, x[:, :DW[1]], 0.0), w1_ref[...],
                   preferred_element_type=jnp.float32)
    m2 = (v >= CUT[2]) & (v < CUT[3])
    acc += jnp.dot(jnp.where(m2, x[:, :DW[2]], 0.0), w2_ref[...],
                   preferred_element_type=jnp.float32)
    m3 = v >= CUT[3]
    acc += jnp.dot(jnp.where(m3, x[:, :DW[3]], 0.0), w3_ref[...],
                   preferred_element_type=jnp.float32)
    o_ref[...] = acc


def _tc_project(ids3, x, w1, w2, w3):
    return pl.pallas_call(
        _tc_body,
        grid=(NT,),
        in_specs=[
            pl.BlockSpec((1, 1, R), lambda i: (i, 0, 0)),
            pl.BlockSpec((R, D_OUT), lambda i: (i, 0)),
            pl.BlockSpec((DW[1], D_OUT), lambda i: (0, 0)),
            pl.BlockSpec((DW[2], D_OUT), lambda i: (0, 0)),
            pl.BlockSpec((DW[3], D_OUT), lambda i: (0, 0)),
        ],
        out_specs=pl.BlockSpec((R, D_OUT), lambda i: (i, 0)),
        out_shape=jax.ShapeDtypeStruct((T, D_OUT), jnp.float32),
    )(ids3, x, w1, w2, w3)


def kernel(input_ids, E0, E1, E2, E3, W1, W2, W3):
    ids = input_ids.reshape(T)
    x = _sc_gather(ids, E0, E1, E2, E3)
    ids3 = ids.reshape(NT, 1, R)
    out = _tc_project(ids3, x, W1, W2, W3)
    return out.reshape(input_ids.shape + (D_OUT,))
